# Initial kernel scaffold; baseline (speedup 1.0000x reference)
import jax
import jax.numpy as jnp
from jax.experimental import pallas as pl


def kernel(x, fe_W1, fe_b1, fe_W2, fe_b2, fn_W1, fn_b1, fn_W2, fn_b2):
    raise NotImplementedError("write your pallas kernel here")


# speedup vs baseline: 16.8896x; 16.8896x over previous
import functools

import jax
import jax.numpy as jnp
from jax.experimental import pallas as pl

_B, _N, _D = 128, 100, 16
_ALPHA = 0.2
_IP = 104


def _leaky(v):
    return jnp.maximum(v, _ALPHA * v)


def _mp_kernel(x_ref, fe_W1_ref, fe_b1_ref, fe_W2_ref, fe_b2_ref,
               fn_W1_ref, fn_b1_ref, fn_W2_ref, fn_b2_ref, o_ref, *, bb):
    W1a = fe_W1_ref[:_D, :]
    W1b = fe_W1_ref[_D:, :]
    fe_b1 = fe_b1_ref[0, :]
    fe_W2 = fe_W2_ref[...]
    fe_b2 = fe_b2_ref[0, :]
    fn_W1 = fn_W1_ref[...]
    fn_b1 = fn_b1_ref[0, :]
    fn_W2 = fn_W2_ref[...]
    fn_b2 = fn_b2_ref[0, :]

    for b in range(bb):
        x2d = x_ref[b]
        P = jnp.dot(x2d, W1a, preferred_element_type=jnp.float32) + fe_b1
        Q = jnp.dot(x2d, W1b, preferred_element_type=jnp.float32)
        P_pad = jnp.concatenate(
            [P, jnp.zeros((_IP - _N, P.shape[1]), jnp.float32)], axis=0)
        E = _leaky(P_pad[None, :, :] + Q[:, None, :])
        E2 = E.reshape(_N * _IP, E.shape[2])
        H = _leaky(jnp.dot(E2, fe_W2, preferred_element_type=jnp.float32)
                   + fe_b2)
        A = H.reshape(_N, _IP, H.shape[1]).sum(axis=0)
        hin = jnp.concatenate([A[:_N, :], x2d], axis=1)
        h = _leaky(jnp.dot(hin, fn_W1, preferred_element_type=jnp.float32)
                   + fn_b1)
        o_ref[b] = (jnp.dot(h, fn_W2, preferred_element_type=jnp.float32)
                    + fn_b2)


def kernel(x, fe_W1, fe_b1, fe_W2, fe_b2, fn_W1, fn_b1, fn_W2, fn_b2):
    bb = 4
    grid = (_B // bb,)
    wspec = lambda r, c: pl.BlockSpec((r, c), lambda i: (0, 0))
    out = pl.pallas_call(
        functools.partial(_mp_kernel, bb=bb),
        grid=grid,
        in_specs=[
            pl.BlockSpec((bb, _N, _D), lambda i: (i, 0, 0)),
            wspec(2 * _D, 32),
            wspec(1, 32),
            wspec(32, 8),
            wspec(1, 8),
            wspec(24, 32),
            wspec(1, 32),
            wspec(32, 16),
            wspec(1, 16),
        ],
        out_specs=pl.BlockSpec((bb, _N, _D), lambda i: (i, 0, 0)),
        out_shape=jax.ShapeDtypeStruct((_B, _N, _D), jnp.float32),
    )(x, fe_W1, fe_b1.reshape(1, -1), fe_W2, fe_b2.reshape(1, -1),
      fn_W1, fn_b1.reshape(1, -1), fn_W2, fn_b2.reshape(1, -1))
    return out


# speedup vs baseline: 29.0415x; 1.7195x over previous
import functools

import jax
import jax.numpy as jnp
from jax.experimental import pallas as pl

_B, _N, _D = 128, 100, 16
_ALPHA = 0.2
_JB = 8
_NL = 128
_F1 = 32
_F2 = 8


def _leaky(v):
    return jnp.maximum(v, _ALPHA * v)


def _mp_kernel(x_ref, W1a_ref, W1b_ref, b1_ref, Wbd_ref, b2s_ref,
               fnW1T_ref, fnb1_ref, fnW2T_ref, fnb2_ref, o_ref, *, bb):
    W1a = W1a_ref[...]
    W1b = W1b_ref[...]
    b1 = b1_ref[0, :]
    Wbd = Wbd_ref[...]
    b2s = b2s_ref[...]
    fnW1T = fnW1T_ref[...]
    fnb1 = fnb1_ref[...]
    fnW2T = fnW2T_ref[...]
    fnb2 = fnb2_ref[...]

    nfull = _N // _JB
    rem = _N - nfull * _JB

    for b in range(bb):
        x2d = x_ref[b]
        P = jnp.dot(x2d, W1a, preferred_element_type=jnp.float32) + b1
        Q = jnp.dot(x2d, W1b, preferred_element_type=jnp.float32)
        Pz = jnp.concatenate(
            [P, jnp.zeros((_NL - _N, _F1), jnp.float32)], axis=0)
        P_T = Pz.T

        acc = jnp.zeros((_JB * _F2, _NL), jnp.float32)
        for jb in range(nfull):
            Qblk = Q[jb * _JB:(jb + 1) * _JB, :]
            Qb3 = jnp.broadcast_to(Qblk[:, :, None], (_JB, _F1, _NL))
            E = _leaky(P_T[None, :, :] + Qb3)
            Es = E.reshape(_JB * _F1, _NL)
            Hs = _leaky(jnp.dot(Wbd, Es, preferred_element_type=jnp.float32)
                        + b2s)
            acc = acc + Hs
        Qblk = Q[nfull * _JB:_N, :]
        Qb3 = jnp.broadcast_to(Qblk[:, :, None], (rem, _F1, _NL))
        E = _leaky(P_T[None, :, :] + Qb3)
        Es = E.reshape(rem * _F1, _NL)
        Hr = _leaky(jnp.dot(Wbd[:rem * _F2, :rem * _F1], Es,
                            preferred_element_type=jnp.float32)
                    + b2s[:rem * _F2, :])

        A_T = (acc.reshape(_JB, _F2, _NL).sum(axis=0)
               + Hr.reshape(rem, _F2, _NL).sum(axis=0))

        xz = jnp.concatenate(
            [x2d, jnp.zeros((_NL - _N, _D), jnp.float32)], axis=0)
        x_T = xz.T
        hin_T = jnp.concatenate([A_T, x_T], axis=0)
        h_T = _leaky(jnp.dot(fnW1T, hin_T,
                             preferred_element_type=jnp.float32) + fnb1)
        o_T = (jnp.dot(fnW2T, h_T, preferred_element_type=jnp.float32)
               + fnb2)
        o_ref[b] = o_T.T[:_N, :]


def kernel(x, fe_W1, fe_b1, fe_W2, fe_b2, fn_W1, fn_b1, fn_W2, fn_b2):
    bb = 4
    grid = (_B // bb,)
    W1a = fe_W1[:_D, :]
    W1b = fe_W1[_D:, :]
    Wbd = jnp.kron(jnp.eye(_JB, dtype=jnp.float32), fe_W2.T)
    b2s = jnp.tile(fe_b2, (_JB,)).reshape(_JB * _F2, 1)
    wspec = lambda r, c: pl.BlockSpec((r, c), lambda i: (0, 0))
    out = pl.pallas_call(
        functools.partial(_mp_kernel, bb=bb),
        grid=grid,
        in_specs=[
            pl.BlockSpec((bb, _N, _D), lambda i: (i, 0, 0)),
            wspec(_D, _F1),
            wspec(_D, _F1),
            wspec(1, _F1),
            wspec(_JB * _F2, _JB * _F1),
            wspec(_JB * _F2, 1),
            wspec(32, 24),
            wspec(32, 1),
            wspec(16, 32),
            wspec(16, 1),
        ],
        out_specs=pl.BlockSpec((bb, _N, _D), lambda i: (i, 0, 0)),
        out_shape=jax.ShapeDtypeStruct((_B, _N, _D), jnp.float32),
    )(x, W1a, W1b, fe_b1.reshape(1, -1), Wbd, b2s,
      fn_W1.T, fn_b1.reshape(-1, 1), fn_W2.T, fn_b2.reshape(-1, 1))
    return out
